# Initial kernel scaffold; baseline (speedup 1.0000x reference)
import jax
import jax.numpy as jnp
from jax.experimental import pallas as pl


def kernel(x, edge_index, neg_idx, W1, b1, W2, b2, Wm1, bm1, gamma, beta, Wm2, bm2, Wp, bp):
    raise NotImplementedError("write your pallas kernel here")


# speedup vs baseline: 1.7484x; 1.7484x over previous
import functools

import jax
import jax.numpy as jnp
from jax import lax
from jax.experimental import pallas as pl
from jax.experimental.pallas import tpu as pltpu
from jax.experimental.pallas import tpu_sc as plsc

N = 10000
E = 320000
K = 32
D = 128
TEMP = 0.5
LAM = 1.0

NC = 2
NS = 16
NW = NC * NS
CHUNK = 256
NCHUNK = 40
PERW = CHUNK * NCHUNK
EPAD = PERW * NW
FCHUNK = 512
NFCHUNK = EPAD // NC // FCHUNK
HALF = EPAD // NC
NPD = 10240
NOUT = 10112
DUMMY = N


@functools.lru_cache(maxsize=None)
def _mesh():
  return plsc.VectorSubcoreMesh(
      core_axis_name="c", subcore_axis_name="s", num_cores=NC,
      num_subcores=NS)


def _wid():
  return lax.axis_index("s") * NC + lax.axis_index("c")


@functools.lru_cache(maxsize=None)
def _make_featsum():
  @functools.partial(
      pl.kernel, mesh=_mesh(),
      compiler_params=pltpu.CompilerParams(use_tc_tiling_on_sc=False),
      out_type=jax.ShapeDtypeStruct((NC, NS, NOUT, 8), jnp.float32),
      scratch_types=[
          pltpu.VMEM((FCHUNK,), jnp.int32),
          pltpu.VMEM((FCHUNK,), jnp.int32),
          pltpu.VMEM((FCHUNK, 8), jnp.float32),
          pltpu.VMEM_SHARED((NS, NOUT, 8), jnp.float32),
          pltpu.SemaphoreType.DMA,
      ],
  )
  def k(tab_hbm, src_hbm, dst_hbm, zeros_hbm, out, idx_s, idx_d, rows_v,
        acc, sem):
    cid = lax.axis_index("c")
    sid = lax.axis_index("s")
    myacc = acc.at[sid]
    pltpu.sync_copy(zeros_hbm, myacc)
    plane = tab_hbm.at[sid]

    def chunk(j, _):
      base = cid * HALF + j * FCHUNK
      pltpu.sync_copy(src_hbm.at[pl.ds(base, FCHUNK)], idx_s)
      pltpu.sync_copy(dst_hbm.at[pl.ds(base, FCHUNK)], idx_d)
      pltpu.async_copy(plane.at[idx_s], rows_v, sem).wait()
      pltpu.sync_copy(rows_v, myacc.at[idx_d], add=True)
      return 0

    lax.fori_loop(0, NFCHUNK, chunk, 0)
    pltpu.sync_copy(myacc, out.at[cid, sid])

  return k


@functools.lru_cache(maxsize=None)
def _make_count():
  out_t = (jax.ShapeDtypeStruct((NC, NOUT, 8), jnp.float32),
           jax.ShapeDtypeStruct((NC, NOUT, 8), jnp.float32))
  rps = NOUT // NS

  @functools.partial(
      pl.kernel, mesh=_mesh(),
      compiler_params=pltpu.CompilerParams(use_tc_tiling_on_sc=False),
      out_type=out_t,
      scratch_types=[
          pltpu.VMEM((CHUNK,), jnp.int32),
          pltpu.VMEM((CHUNK, 8), jnp.float32),
          pltpu.VMEM_SHARED((NOUT, 8), jnp.float32),
          pltpu.VMEM_SHARED((NOUT, 8), jnp.float32),
      ],
  )
  def k(src_hbm, dst_hbm, zeros_hbm, ones_hbm, out_s, out_d, idx_v, ones_v,
        acc_s, acc_d):
    cid = lax.axis_index("c")
    sid = lax.axis_index("s")
    w = _wid()
    pltpu.sync_copy(ones_hbm, ones_v)
    rows = pl.ds(sid * rps, rps)
    pltpu.sync_copy(zeros_hbm.at[rows], acc_s.at[rows])
    pltpu.sync_copy(zeros_hbm.at[rows], acc_d.at[rows])
    plsc.subcore_barrier()

    def chunk(j, _):
      base = w * PERW + j * CHUNK
      pltpu.sync_copy(src_hbm.at[pl.ds(base, CHUNK)], idx_v)
      pltpu.sync_copy(ones_v, acc_s.at[idx_v], add=True)
      pltpu.sync_copy(dst_hbm.at[pl.ds(base, CHUNK)], idx_v)
      pltpu.sync_copy(ones_v, acc_d.at[idx_v], add=True)
      return 0

    lax.fori_loop(0, NCHUNK, chunk, 0)
    plsc.subcore_barrier()
    pltpu.sync_copy(acc_s.at[rows], out_s.at[cid, rows])
    pltpu.sync_copy(acc_d.at[rows], out_d.at[cid, rows])

  return k


@functools.lru_cache(maxsize=None)
def _make_scatter8():
  rps = NOUT // NS

  @functools.partial(
      pl.kernel, mesh=_mesh(),
      compiler_params=pltpu.CompilerParams(use_tc_tiling_on_sc=False),
      out_type=jax.ShapeDtypeStruct((NC, NOUT, 8), jnp.float32),
      scratch_types=[
          pltpu.VMEM((CHUNK,), jnp.int32),
          pltpu.VMEM((CHUNK, 8), jnp.float32),
          pltpu.VMEM_SHARED((NOUT, 8), jnp.float32),
      ],
  )
  def k(val_hbm, dst_hbm, zeros_hbm, out, idx_v, rows_v, acc):
    cid = lax.axis_index("c")
    sid = lax.axis_index("s")
    w = _wid()
    rows = pl.ds(sid * rps, rps)
    pltpu.sync_copy(zeros_hbm.at[rows], acc.at[rows])
    plsc.subcore_barrier()

    def chunk(j, _):
      base = w * PERW + j * CHUNK
      pltpu.sync_copy(dst_hbm.at[pl.ds(base, CHUNK)], idx_v)
      pltpu.sync_copy(val_hbm.at[pl.ds(base, CHUNK)], rows_v)
      pltpu.sync_copy(rows_v, acc.at[idx_v], add=True)
      return 0

    lax.fori_loop(0, NCHUNK, chunk, 0)
    plsc.subcore_barrier()
    pltpu.sync_copy(acc.at[rows], out.at[cid, rows])

  return k


@functools.lru_cache(maxsize=None)
def _make_gather2(d1, d2):
  out_t = (jax.ShapeDtypeStruct((EPAD, d1), jnp.float32),
           jax.ShapeDtypeStruct((EPAD, d2), jnp.float32))

  @functools.partial(
      pl.kernel, mesh=_mesh(),
      compiler_params=pltpu.CompilerParams(use_tc_tiling_on_sc=False), out_type=out_t,
      scratch_types=[
          pltpu.VMEM((CHUNK,), jnp.int32),
          pltpu.VMEM((CHUNK, d1), jnp.float32),
          pltpu.VMEM((CHUNK, d2), jnp.float32),
          pltpu.SemaphoreType.DMA,
      ],
  )
  def k(tab1, tab2, idx_hbm, out1, out2, idx_v, r1, r2, sem):
    w = _wid()

    def chunk(j, _):
      base = w * PERW + j * CHUNK
      pltpu.sync_copy(idx_hbm.at[pl.ds(base, CHUNK)], idx_v)
      pltpu.async_copy(tab1.at[idx_v], r1, sem).wait()
      pltpu.async_copy(tab2.at[idx_v], r2, sem).wait()
      pltpu.sync_copy(r1, out1.at[pl.ds(base, CHUNK)])
      pltpu.sync_copy(r2, out2.at[pl.ds(base, CHUNK)])
      return 0

    lax.fori_loop(0, NCHUNK, chunk, 0)

  return k


@functools.lru_cache(maxsize=None)
def _make_gather1(d1):
  @functools.partial(
      pl.kernel, mesh=_mesh(),
      compiler_params=pltpu.CompilerParams(use_tc_tiling_on_sc=False),
      out_type=jax.ShapeDtypeStruct((EPAD, d1), jnp.float32),
      scratch_types=[
          pltpu.VMEM((CHUNK,), jnp.int32),
          pltpu.VMEM((CHUNK, d1), jnp.float32),
          pltpu.SemaphoreType.DMA,
      ],
  )
  def k(tab, idx_hbm, out, idx_v, r1, sem):
    w = _wid()

    def chunk(j, _):
      base = w * PERW + j * CHUNK
      pltpu.sync_copy(idx_hbm.at[pl.ds(base, CHUNK)], idx_v)
      pltpu.async_copy(tab.at[idx_v], r1, sem).wait()
      pltpu.sync_copy(r1, out.at[pl.ds(base, CHUNK)])
      return 0

    lax.fori_loop(0, NCHUNK, chunk, 0)

  return k


def _tc_edge_prep(sp, dp):
  def body(s_ref, d_ref, sg_ref, dg_ref, se_ref, de_ref):
    s = s_ref[...]
    d = d_ref[...]
    valid = jnp.logical_and(s != d, s < N)
    sg_ref[...] = jnp.minimum(s, N - 1)
    dg_ref[...] = jnp.minimum(d, N - 1)
    se_ref[...] = jnp.where(valid, s, DUMMY)
    de_ref[...] = jnp.where(valid, d, DUMMY)

  o = jax.ShapeDtypeStruct((640, 512), jnp.int32)
  return pl.pallas_call(body, out_shape=(o, o, o, o))(
      sp.reshape(640, 512), dp.reshape(640, 512))


def _tc_sum_planes(parts):
  nb = 632

  def body(p_ref, out_ref):
    out_ref[...] = p_ref[0] + p_ref[1]

  return pl.pallas_call(
      body,
      grid=(NOUT // nb,),
      in_specs=[pl.BlockSpec((NC, nb, 8), lambda i: (0, i, 0))],
      out_specs=pl.BlockSpec((nb, 8), lambda i: (i, 0)),
      out_shape=jax.ShapeDtypeStruct((NOUT, 8), jnp.float32),
  )(parts)


def _l2n(a):
  nrm = jnp.sqrt(jnp.sum(a * a, axis=1, keepdims=True))
  return a / jnp.maximum(nrm, 1e-12)


def _tc_dense_pre(x, wm1, bm1, gamma, beta, wm2, bm2, wp, bp, co8, ci8):
  def body(x_ref, wm1_ref, bm1_ref, g_ref, b_ref, wm2_ref, bm2_ref, wp_ref,
           bp_ref, co_ref, ci_ref, qn_ref, h21_ref, ns_ref, nd_ref, di_ref):
    x_v = x_ref[...]
    t = jnp.dot(x_v, wm1_ref[...], preferred_element_type=jnp.float32)
    t = t + bm1_ref[...]
    mu = jnp.mean(t, axis=0, keepdims=True)
    var = jnp.mean((t - mu) * (t - mu), axis=0, keepdims=True)
    t = (t - mu) * lax.rsqrt(var + 1e-5) * g_ref[...] + b_ref[...]
    trans = jnp.dot(jnp.maximum(t, 0.0), wm2_ref[...],
                    preferred_element_type=jnp.float32) + bm2_ref[...]
    q = jnp.dot(trans, wp_ref[...],
                preferred_element_type=jnp.float32) + bp_ref[...]
    qn_ref[...] = _l2n(q)
    deg_o = 1.0 + co_ref[:, 0:1]
    deg_i = 1.0 + ci_ref[:, 0:1]
    ns = lax.rsqrt(deg_o)
    ns_ref[...] = jnp.broadcast_to(ns, (N, 8))
    nd_ref[...] = jnp.broadcast_to(lax.rsqrt(deg_i), (N, 8))
    di_ref[...] = jnp.broadcast_to(deg_i, (N, 8))
    h21_ref[...] = x_v * ns

  o = jax.ShapeDtypeStruct((N, D), jnp.float32)
  o8 = jax.ShapeDtypeStruct((N, 8), jnp.float32)
  return pl.pallas_call(body, out_shape=(o, o, o8, o8, o8))(
      x, wm1, bm1, gamma, beta, wm2, bm2, wp, bp, co8, ci8)


def _tc_conv_finish1(aggp, h2, nd8, ns8, w1, b1):
  def body(a_ref, h2_ref, nd_ref, ns_ref, w_ref, b_ref, out_ref):
    agg = (a_ref[0] + a_ref[1] + h2_ref[...]) * nd_ref[:, 0:1]
    h = jnp.dot(agg, w_ref[...], preferred_element_type=jnp.float32)
    h = jnp.maximum(h + b_ref[...], 0.0)
    out_ref[...] = h * ns_ref[:, 0:1]

  o = jax.ShapeDtypeStruct((N, D), jnp.float32)
  return pl.pallas_call(body, out_shape=o)(aggp, h2, nd8, ns8, w1, b1)


def _tc_conv_finish2(aggp, h2, nd8, w2, b2):
  def body(a_ref, h2_ref, nd_ref, w_ref, b_ref, z_ref):
    agg = (a_ref[0] + a_ref[1] + h2_ref[...]) * nd_ref[:, 0:1]
    h = jnp.dot(agg, w_ref[...], preferred_element_type=jnp.float32)
    z_ref[...] = _l2n(h + b_ref[...])

  o = jax.ShapeDtypeStruct((N, D), jnp.float32)
  return pl.pallas_call(body, out_shape=o)(aggp, h2, nd8, w2, b2)


def _tc_neg_node(gzn, gqn, zp):
  nb = 128

  def body(gz_ref, gq_ref, z_ref, zs_ref, c_ref):
    gz = gz_ref[...]
    gq = gq_ref[...]
    z_v = z_ref[...]
    zs_ref[...] = jnp.sum(gz, axis=1)
    dots = lax.dot_general(z_v, gq, (((1,), (2,)), ((0,), (0,))),
                           preferred_element_type=jnp.float32)
    c = LAM * jnp.sum(jnp.exp(dots / TEMP), axis=1, keepdims=True)
    c_ref[...] = jnp.broadcast_to(c, (nb, 16))

  grid = NPD // nb
  return pl.pallas_call(
      body,
      grid=(grid,),
      in_specs=[
          pl.BlockSpec((nb, K, D), lambda i: (i, 0, 0)),
          pl.BlockSpec((nb, K, D), lambda i: (i, 0, 0)),
          pl.BlockSpec((nb, D), lambda i: (i, 0)),
      ],
      out_specs=(
          pl.BlockSpec((nb, D), lambda i: (i, 0)),
          pl.BlockSpec((nb, 16), lambda i: (i, 0)),
      ),
      out_shape=(jax.ShapeDtypeStruct((NPD, D), jnp.float32),
                 jax.ShapeDtypeStruct((NPD, 16), jnp.float32)),
  )(gzn, gqn, zp)


def _tc_edge_score(gz, gzs, gc):
  eb = 8192

  def body(gz_ref, gzs_ref, gc_ref, m_ref):
    raw = jnp.sum(gz_ref[...] * gzs_ref[...], axis=1, keepdims=True)
    raw = raw + gc_ref[:, 0:1]
    m_ref[...] = jnp.broadcast_to(jnp.log(raw), (eb, 8))

  return pl.pallas_call(
      body,
      grid=(EPAD // eb,),
      in_specs=[
          pl.BlockSpec((eb, D), lambda i: (i, 0)),
          pl.BlockSpec((eb, D), lambda i: (i, 0)),
          pl.BlockSpec((eb, 16), lambda i: (i, 0)),
      ],
      out_specs=pl.BlockSpec((eb, 8), lambda i: (i, 0)),
      out_shape=jax.ShapeDtypeStruct((EPAD, 8), jnp.float32),
  )(gz, gzs, gc)


def _tc_final(msum8, aggzp, z, qn, zs_n, c_n, di8):
  def body(m_ref, az_ref, z_ref, qn_ref, zs_ref, c_ref, di_ref, out_ref):
    z_v = z_ref[...]
    deg = di_ref[:, 0:1]
    aggz = az_ref[0] + az_ref[1] + z_v
    pos = jnp.sum(aggz * qn_ref[...], axis=1, keepdims=True) / TEMP / deg
    selfm = jnp.log(
        jnp.sum(z_v * zs_ref[...], axis=1, keepdims=True) + c_ref[:, 0:1])
    neg = (m_ref[:, 0:1] + selfm) / deg
    out_ref[...] = neg - pos

  o = jax.ShapeDtypeStruct((N, 1), jnp.float32)
  return pl.pallas_call(body, out_shape=o)(
      msum8, aggzp, z, qn, zs_n, c_n, di8)


@jax.jit
def kernel(x, edge_index, neg_idx, W1, b1, W2, b2, Wm1, bm1, gamma, beta,
           Wm2, bm2, Wp, bp):
  featsum_k = _make_featsum()
  count_k = _make_count()
  scatter8_k = _make_scatter8()
  gather_zq_k = _make_gather2(D, D)
  gather_zc_k = _make_gather2(D, 16)
  gather_zs_k = _make_gather1(D)

  r1 = lambda v: v.reshape(1, D)
  zeros8 = jnp.zeros((NOUT, 8), jnp.float32)
  ones8 = jnp.ones((CHUNK, 8), jnp.float32)
  pad = jnp.full((EPAD - E,), N, jnp.int32)
  sp = jnp.concatenate([edge_index[0], pad])
  dp = jnp.concatenate([edge_index[1], pad])

  src_g, dst_g, src_eff, dst_eff = _tc_edge_prep(sp, dp)
  src_g = src_g.reshape(EPAD)
  dst_g = dst_g.reshape(EPAD)
  src_eff = src_eff.reshape(EPAD)
  dst_eff = dst_eff.reshape(EPAD)

  cnt_out, cnt_in = count_k(src_eff, dst_eff, zeros8, ones8)
  co8 = _tc_sum_planes(cnt_out)[:N]
  ci8 = _tc_sum_planes(cnt_in)[:N]

  cg = lambda v: v.reshape(N, 16, 8).transpose(1, 0, 2)
  uncg = lambda p: p.transpose(0, 2, 1, 3).reshape(NC, NOUT, D)[:, :N]

  qn, h21, ns8, nd8, di8 = _tc_dense_pre(
      x, Wm1, r1(bm1), r1(gamma), r1(beta), Wm2, r1(bm2), Wp, r1(bp),
      co8, ci8)

  agg1 = uncg(featsum_k(cg(h21), src_g, dst_eff, zeros8))
  h22 = _tc_conv_finish1(agg1, h21, nd8, ns8, W1, r1(b1))
  agg2 = uncg(featsum_k(cg(h22), src_g, dst_eff, zeros8))
  z = _tc_conv_finish2(agg2, h22, nd8, W2, r1(b2))

  neg_flat = jnp.concatenate(
      [neg_idx.reshape(-1), jnp.zeros((EPAD - N * K,), jnp.int32)])
  gz_neg, gq_neg = gather_zq_k(z, qn, neg_flat)
  zp = jnp.concatenate([z, jnp.zeros((NPD - N, D), jnp.float32)])
  zs_sum, c16 = _tc_neg_node(
      gz_neg.reshape(NPD, K, D), gq_neg.reshape(NPD, K, D), zp)

  gz_e, gc_e = gather_zc_k(z, c16, src_g)
  gzs_e = gather_zs_k(zs_sum, dst_g)
  m8 = _tc_edge_score(gz_e, gzs_e, gc_e)
  msum = scatter8_k(m8, dst_eff, zeros8)
  msum8 = _tc_sum_planes(msum)[:N]

  aggz = uncg(featsum_k(cg(z), src_g, dst_eff, zeros8))
  out = _tc_final(msum8, aggz, z, qn, zs_sum[:N], c16[:N], di8)
  return out.reshape(N)


# speedup vs baseline: 1.9009x; 1.0872x over previous
import functools

import jax
import jax.numpy as jnp
from jax import lax
from jax.experimental import pallas as pl
from jax.experimental.pallas import tpu as pltpu
from jax.experimental.pallas import tpu_sc as plsc

N = 10000
E = 320000
K = 32
D = 128
TEMP = 0.5
LAM = 1.0

NC = 2
NS = 16
NW = NC * NS
CHUNK = 256
NCHUNK = 40
PERW = CHUNK * NCHUNK
EPAD = PERW * NW
FCHUNK = 512
NFCHUNK = EPAD // NC // FCHUNK
HALF = EPAD // NC
NPD = 10240
NOUT = 10112
DUMMY = N


@functools.lru_cache(maxsize=None)
def _mesh():
  return plsc.VectorSubcoreMesh(
      core_axis_name="c", subcore_axis_name="s", num_cores=NC,
      num_subcores=NS)


def _wid():
  return lax.axis_index("s") * NC + lax.axis_index("c")


@functools.lru_cache(maxsize=None)
def _make_featsum():
  @functools.partial(
      pl.kernel, mesh=_mesh(),
      compiler_params=pltpu.CompilerParams(use_tc_tiling_on_sc=False),
      out_type=jax.ShapeDtypeStruct((NC, NS, NOUT, 8), jnp.float32),
      scratch_types=[
          pltpu.VMEM((FCHUNK,), jnp.int32),
          pltpu.VMEM((FCHUNK,), jnp.int32),
          pltpu.VMEM((FCHUNK, 8), jnp.float32),
          pltpu.VMEM_SHARED((NS, NOUT, 8), jnp.float32),
          pltpu.SemaphoreType.DMA,
      ],
  )
  def k(tab_hbm, src_hbm, dst_hbm, zeros_hbm, out, idx_s, idx_d, rows_v,
        acc, sem):
    cid = lax.axis_index("c")
    sid = lax.axis_index("s")
    myacc = acc.at[sid]
    pltpu.sync_copy(zeros_hbm, myacc)
    plane = tab_hbm.at[sid]

    def chunk(j, _):
      base = cid * HALF + j * FCHUNK
      pltpu.sync_copy(src_hbm.at[pl.ds(base, FCHUNK)], idx_s)
      pltpu.sync_copy(dst_hbm.at[pl.ds(base, FCHUNK)], idx_d)
      pltpu.async_copy(plane.at[idx_s], rows_v, sem).wait()
      pltpu.sync_copy(rows_v, myacc.at[idx_d], add=True)
      return 0

    lax.fori_loop(0, NFCHUNK, chunk, 0)
    pltpu.sync_copy(myacc, out.at[cid, sid])

  return k


@functools.lru_cache(maxsize=None)
def _make_count():
  out_t = (jax.ShapeDtypeStruct((NC, NOUT, 8), jnp.float32),
           jax.ShapeDtypeStruct((NC, NOUT, 8), jnp.float32))
  rps = NOUT // NS

  @functools.partial(
      pl.kernel, mesh=_mesh(),
      compiler_params=pltpu.CompilerParams(use_tc_tiling_on_sc=False),
      out_type=out_t,
      scratch_types=[
          pltpu.VMEM((CHUNK,), jnp.int32),
          pltpu.VMEM((CHUNK, 8), jnp.float32),
          pltpu.VMEM_SHARED((NOUT, 8), jnp.float32),
          pltpu.VMEM_SHARED((NOUT, 8), jnp.float32),
      ],
  )
  def k(src_hbm, dst_hbm, zeros_hbm, ones_hbm, out_s, out_d, idx_v, ones_v,
        acc_s, acc_d):
    cid = lax.axis_index("c")
    sid = lax.axis_index("s")
    w = _wid()
    pltpu.sync_copy(ones_hbm, ones_v)
    rows = pl.ds(sid * rps, rps)
    pltpu.sync_copy(zeros_hbm.at[rows], acc_s.at[rows])
    pltpu.sync_copy(zeros_hbm.at[rows], acc_d.at[rows])
    plsc.subcore_barrier()

    def chunk(j, _):
      base = w * PERW + j * CHUNK
      pltpu.sync_copy(src_hbm.at[pl.ds(base, CHUNK)], idx_v)
      pltpu.sync_copy(ones_v, acc_s.at[idx_v], add=True)
      pltpu.sync_copy(dst_hbm.at[pl.ds(base, CHUNK)], idx_v)
      pltpu.sync_copy(ones_v, acc_d.at[idx_v], add=True)
      return 0

    lax.fori_loop(0, NCHUNK, chunk, 0)
    plsc.subcore_barrier()
    pltpu.sync_copy(acc_s.at[rows], out_s.at[cid, rows])
    pltpu.sync_copy(acc_d.at[rows], out_d.at[cid, rows])

  return k


@functools.lru_cache(maxsize=None)
def _make_scatter8():
  rps = NOUT // NS

  @functools.partial(
      pl.kernel, mesh=_mesh(),
      compiler_params=pltpu.CompilerParams(use_tc_tiling_on_sc=False),
      out_type=jax.ShapeDtypeStruct((NC, NOUT, 8), jnp.float32),
      scratch_types=[
          pltpu.VMEM((CHUNK,), jnp.int32),
          pltpu.VMEM((CHUNK, 8), jnp.float32),
          pltpu.VMEM_SHARED((NOUT, 8), jnp.float32),
      ],
  )
  def k(val_hbm, dst_hbm, zeros_hbm, out, idx_v, rows_v, acc):
    cid = lax.axis_index("c")
    sid = lax.axis_index("s")
    w = _wid()
    rows = pl.ds(sid * rps, rps)
    pltpu.sync_copy(zeros_hbm.at[rows], acc.at[rows])
    plsc.subcore_barrier()

    def chunk(j, _):
      base = w * PERW + j * CHUNK
      pltpu.sync_copy(dst_hbm.at[pl.ds(base, CHUNK)], idx_v)
      pltpu.sync_copy(val_hbm.at[pl.ds(base, CHUNK)], rows_v)
      pltpu.sync_copy(rows_v, acc.at[idx_v], add=True)
      return 0

    lax.fori_loop(0, NCHUNK, chunk, 0)
    plsc.subcore_barrier()
    pltpu.sync_copy(acc.at[rows], out.at[cid, rows])

  return k


@functools.lru_cache(maxsize=None)
def _make_gather2(d1, d2):
  out_t = (jax.ShapeDtypeStruct((EPAD, d1), jnp.float32),
           jax.ShapeDtypeStruct((EPAD, d2), jnp.float32))

  gchunk = 128
  gn = PERW // gchunk

  @functools.partial(
      pl.kernel, mesh=_mesh(),
      compiler_params=pltpu.CompilerParams(use_tc_tiling_on_sc=False), out_type=out_t,
      scratch_types=[
          pltpu.VMEM((gchunk,), jnp.int32),
          pltpu.VMEM((gchunk,), jnp.int32),
          pltpu.VMEM((gchunk, d1), jnp.float32),
          pltpu.VMEM((gchunk, d1), jnp.float32),
          pltpu.VMEM((gchunk, d2), jnp.float32),
          pltpu.VMEM((gchunk, d2), jnp.float32),
          pltpu.SemaphoreType.DMA,
          pltpu.SemaphoreType.DMA,
          pltpu.SemaphoreType.DMA,
          pltpu.SemaphoreType.DMA,
      ],
  )
  def k(tab1, tab2, idx_hbm, out1, out2, idx_v, idx_v2, r1, r1b, r2, r2b,
        sem, sem2, sem3, sem4):
    w = _wid()

    def chunk(jj, _):
      base0 = w * PERW + (2 * jj) * gchunk
      pltpu.sync_copy(idx_hbm.at[pl.ds(base0, gchunk)], idx_v)
      g1 = pltpu.async_copy(tab1.at[idx_v], r1, sem)
      g2 = pltpu.async_copy(tab2.at[idx_v], r2, sem2)
      g1.wait()
      w1 = pltpu.async_copy(r1, out1.at[pl.ds(base0, gchunk)], sem3)
      g2.wait()
      w2 = pltpu.async_copy(r2, out2.at[pl.ds(base0, gchunk)], sem4)
      base1 = base0 + gchunk
      pltpu.sync_copy(idx_hbm.at[pl.ds(base1, gchunk)], idx_v2)
      g3 = pltpu.async_copy(tab1.at[idx_v2], r1b, sem)
      g4 = pltpu.async_copy(tab2.at[idx_v2], r2b, sem2)
      g3.wait()
      w3 = pltpu.async_copy(r1b, out1.at[pl.ds(base1, gchunk)], sem3)
      g4.wait()
      w4 = pltpu.async_copy(r2b, out2.at[pl.ds(base1, gchunk)], sem4)
      w1.wait()
      w2.wait()
      w3.wait()
      w4.wait()
      return 0

    lax.fori_loop(0, gn // 2, chunk, 0)

  return k


@functools.lru_cache(maxsize=None)
def _make_gather1(d1):
  @functools.partial(
      pl.kernel, mesh=_mesh(),
      compiler_params=pltpu.CompilerParams(use_tc_tiling_on_sc=False),
      out_type=jax.ShapeDtypeStruct((EPAD, d1), jnp.float32),
      scratch_types=[
          pltpu.VMEM((CHUNK,), jnp.int32),
          pltpu.VMEM((CHUNK,), jnp.int32),
          pltpu.VMEM((CHUNK, d1), jnp.float32),
          pltpu.VMEM((CHUNK, d1), jnp.float32),
          pltpu.SemaphoreType.DMA,
          pltpu.SemaphoreType.DMA,
          pltpu.SemaphoreType.DMA,
      ],
  )
  def k(tab, idx_hbm, out, idx_v, idx_v2, r1, r1b, sem, sem2, sem3):
    w = _wid()

    def chunk(jj, _):
      base0 = w * PERW + (2 * jj) * CHUNK
      pltpu.sync_copy(idx_hbm.at[pl.ds(base0, CHUNK)], idx_v)
      pltpu.async_copy(tab.at[idx_v], r1, sem).wait()
      w1 = pltpu.async_copy(r1, out.at[pl.ds(base0, CHUNK)], sem2)
      base1 = base0 + CHUNK
      pltpu.sync_copy(idx_hbm.at[pl.ds(base1, CHUNK)], idx_v2)
      pltpu.async_copy(tab.at[idx_v2], r1b, sem).wait()
      w2 = pltpu.async_copy(r1b, out.at[pl.ds(base1, CHUNK)], sem3)
      w1.wait()
      w2.wait()
      return 0

    lax.fori_loop(0, NCHUNK // 2, chunk, 0)

  return k


def _tc_edge_prep(sp, dp):
  def body(s_ref, d_ref, sg_ref, dg_ref, se_ref, de_ref):
    s = s_ref[...]
    d = d_ref[...]
    valid = jnp.logical_and(s != d, s < N)
    sg_ref[...] = jnp.minimum(s, N - 1)
    dg_ref[...] = jnp.minimum(d, N - 1)
    se_ref[...] = jnp.where(valid, s, DUMMY)
    de_ref[...] = jnp.where(valid, d, DUMMY)

  o = jax.ShapeDtypeStruct((640, 512), jnp.int32)
  return pl.pallas_call(body, out_shape=(o, o, o, o))(
      sp.reshape(640, 512), dp.reshape(640, 512))


def _tc_sum_planes(parts):
  nb = 632

  def body(p_ref, out_ref):
    out_ref[...] = p_ref[0] + p_ref[1]

  return pl.pallas_call(
      body,
      grid=(NOUT // nb,),
      in_specs=[pl.BlockSpec((NC, nb, 8), lambda i: (0, i, 0))],
      out_specs=pl.BlockSpec((nb, 8), lambda i: (i, 0)),
      out_shape=jax.ShapeDtypeStruct((NOUT, 8), jnp.float32),
  )(parts)


def _l2n(a):
  nrm = jnp.sqrt(jnp.sum(a * a, axis=1, keepdims=True))
  return a / jnp.maximum(nrm, 1e-12)


def _tc_dense_pre(x, wm1, bm1, gamma, beta, wm2, bm2, wp, bp, co8, ci8):
  def body(x_ref, wm1_ref, bm1_ref, g_ref, b_ref, wm2_ref, bm2_ref, wp_ref,
           bp_ref, co_ref, ci_ref, qn_ref, h21_ref, ns_ref, nd_ref, di_ref):
    x_v = x_ref[...]
    t = jnp.dot(x_v, wm1_ref[...], preferred_element_type=jnp.float32)
    t = t + bm1_ref[...]
    mu = jnp.mean(t, axis=0, keepdims=True)
    var = jnp.mean((t - mu) * (t - mu), axis=0, keepdims=True)
    t = (t - mu) * lax.rsqrt(var + 1e-5) * g_ref[...] + b_ref[...]
    trans = jnp.dot(jnp.maximum(t, 0.0), wm2_ref[...],
                    preferred_element_type=jnp.float32) + bm2_ref[...]
    q = jnp.dot(trans, wp_ref[...],
                preferred_element_type=jnp.float32) + bp_ref[...]
    qn_ref[...] = _l2n(q)
    deg_o = 1.0 + co_ref[:, 0:1]
    deg_i = 1.0 + ci_ref[:, 0:1]
    ns = lax.rsqrt(deg_o)
    ns_ref[...] = jnp.broadcast_to(ns, (N, 8))
    nd_ref[...] = jnp.broadcast_to(lax.rsqrt(deg_i), (N, 8))
    di_ref[...] = jnp.broadcast_to(deg_i, (N, 8))
    h21_ref[...] = x_v * ns

  o = jax.ShapeDtypeStruct((N, D), jnp.float32)
  o8 = jax.ShapeDtypeStruct((N, 8), jnp.float32)
  return pl.pallas_call(body, out_shape=(o, o, o8, o8, o8))(
      x, wm1, bm1, gamma, beta, wm2, bm2, wp, bp, co8, ci8)


def _tc_conv_finish1(aggp, h2, nd8, ns8, w1, b1):
  def body(a_ref, h2_ref, nd_ref, ns_ref, w_ref, b_ref, out_ref):
    agg = (a_ref[0] + a_ref[1] + h2_ref[...]) * nd_ref[:, 0:1]
    h = jnp.dot(agg, w_ref[...], preferred_element_type=jnp.float32)
    h = jnp.maximum(h + b_ref[...], 0.0)
    out_ref[...] = h * ns_ref[:, 0:1]

  o = jax.ShapeDtypeStruct((N, D), jnp.float32)
  return pl.pallas_call(body, out_shape=o)(aggp, h2, nd8, ns8, w1, b1)


def _tc_conv_finish2(aggp, h2, nd8, w2, b2):
  def body(a_ref, h2_ref, nd_ref, w_ref, b_ref, z_ref):
    agg = (a_ref[0] + a_ref[1] + h2_ref[...]) * nd_ref[:, 0:1]
    h = jnp.dot(agg, w_ref[...], preferred_element_type=jnp.float32)
    z_ref[...] = _l2n(h + b_ref[...])

  o = jax.ShapeDtypeStruct((N, D), jnp.float32)
  return pl.pallas_call(body, out_shape=o)(aggp, h2, nd8, w2, b2)


def _tc_neg_node(gzn, gqn, zp):
  nb = 128

  def body(gz_ref, gq_ref, z_ref, zs_ref, c_ref):
    gz = gz_ref[...]
    gq = gq_ref[...]
    z_v = z_ref[...]
    zs_ref[...] = jnp.sum(gz, axis=1)
    dots = lax.dot_general(z_v, gq, (((1,), (2,)), ((0,), (0,))),
                           preferred_element_type=jnp.float32)
    c = LAM * jnp.sum(jnp.exp(dots / TEMP), axis=1, keepdims=True)
    c_ref[...] = jnp.broadcast_to(c, (nb, 16))

  grid = NPD // nb
  return pl.pallas_call(
      body,
      grid=(grid,),
      in_specs=[
          pl.BlockSpec((nb, K, D), lambda i: (i, 0, 0)),
          pl.BlockSpec((nb, K, D), lambda i: (i, 0, 0)),
          pl.BlockSpec((nb, D), lambda i: (i, 0)),
      ],
      out_specs=(
          pl.BlockSpec((nb, D), lambda i: (i, 0)),
          pl.BlockSpec((nb, 16), lambda i: (i, 0)),
      ),
      out_shape=(jax.ShapeDtypeStruct((NPD, D), jnp.float32),
                 jax.ShapeDtypeStruct((NPD, 16), jnp.float32)),
  )(gzn, gqn, zp)


def _tc_edge_score(gz, gzs, gc):
  eb = 8192

  def body(gz_ref, gzs_ref, gc_ref, m_ref):
    raw = jnp.sum(gz_ref[...] * gzs_ref[...], axis=1, keepdims=True)
    raw = raw + gc_ref[:, 0:1]
    m_ref[...] = jnp.broadcast_to(jnp.log(raw), (eb, 8))

  return pl.pallas_call(
      body,
      grid=(EPAD // eb,),
      in_specs=[
          pl.BlockSpec((eb, D), lambda i: (i, 0)),
          pl.BlockSpec((eb, D), lambda i: (i, 0)),
          pl.BlockSpec((eb, 16), lambda i: (i, 0)),
      ],
      out_specs=pl.BlockSpec((eb, 8), lambda i: (i, 0)),
      out_shape=jax.ShapeDtypeStruct((EPAD, 8), jnp.float32),
  )(gz, gzs, gc)


def _tc_final(msum8, aggzp, z, qn, zs_n, c_n, di8):
  def body(m_ref, az_ref, z_ref, qn_ref, zs_ref, c_ref, di_ref, out_ref):
    z_v = z_ref[...]
    deg = di_ref[:, 0:1]
    aggz = az_ref[0] + az_ref[1] + z_v
    pos = jnp.sum(aggz * qn_ref[...], axis=1, keepdims=True) / TEMP / deg
    selfm = jnp.log(
        jnp.sum(z_v * zs_ref[...], axis=1, keepdims=True) + c_ref[:, 0:1])
    neg = (m_ref[:, 0:1] + selfm) / deg
    out_ref[...] = neg - pos

  o = jax.ShapeDtypeStruct((N, 1), jnp.float32)
  return pl.pallas_call(body, out_shape=o)(
      msum8, aggzp, z, qn, zs_n, c_n, di8)


@jax.jit
def kernel(x, edge_index, neg_idx, W1, b1, W2, b2, Wm1, bm1, gamma, beta,
           Wm2, bm2, Wp, bp):
  featsum_k = _make_featsum()
  count_k = _make_count()
  scatter8_k = _make_scatter8()
  gather_zq_k = _make_gather2(D, D)
  gather_zc_k = _make_gather2(D, 16)
  gather_zs_k = _make_gather1(D)

  r1 = lambda v: v.reshape(1, D)
  zeros8 = jnp.zeros((NOUT, 8), jnp.float32)
  ones8 = jnp.ones((CHUNK, 8), jnp.float32)
  pad = jnp.full((EPAD - E,), N, jnp.int32)
  sp = jnp.concatenate([edge_index[0], pad])
  dp = jnp.concatenate([edge_index[1], pad])

  src_g, dst_g, src_eff, dst_eff = _tc_edge_prep(sp, dp)
  src_g = src_g.reshape(EPAD)
  dst_g = dst_g.reshape(EPAD)
  src_eff = src_eff.reshape(EPAD)
  dst_eff = dst_eff.reshape(EPAD)

  cnt_out, cnt_in = count_k(src_eff, dst_eff, zeros8, ones8)
  co8 = _tc_sum_planes(cnt_out)[:N]
  ci8 = _tc_sum_planes(cnt_in)[:N]

  cg = lambda v: v.reshape(N, 16, 8).transpose(1, 0, 2)
  uncg = lambda p: p.transpose(0, 2, 1, 3).reshape(NC, NOUT, D)[:, :N]

  qn, h21, ns8, nd8, di8 = _tc_dense_pre(
      x, Wm1, r1(bm1), r1(gamma), r1(beta), Wm2, r1(bm2), Wp, r1(bp),
      co8, ci8)

  agg1 = uncg(featsum_k(cg(h21), src_g, dst_eff, zeros8))
  h22 = _tc_conv_finish1(agg1, h21, nd8, ns8, W1, r1(b1))
  agg2 = uncg(featsum_k(cg(h22), src_g, dst_eff, zeros8))
  z = _tc_conv_finish2(agg2, h22, nd8, W2, r1(b2))

  neg_flat = jnp.concatenate(
      [neg_idx.reshape(-1), jnp.zeros((EPAD - N * K,), jnp.int32)])
  gz_neg, gq_neg = gather_zq_k(z, qn, neg_flat)
  zp = jnp.concatenate([z, jnp.zeros((NPD - N, D), jnp.float32)])
  zs_sum, c16 = _tc_neg_node(
      gz_neg.reshape(NPD, K, D), gq_neg.reshape(NPD, K, D), zp)

  gz_e, gc_e = gather_zc_k(z, c16, src_g)
  gzs_e = gather_zs_k(zs_sum, dst_g)
  m8 = _tc_edge_score(gz_e, gzs_e, gc_e)
  msum = scatter8_k(m8, dst_eff, zeros8)
  msum8 = _tc_sum_planes(msum)[:N]

  aggz = uncg(featsum_k(cg(z), src_g, dst_eff, zeros8))
  out = _tc_final(msum8, aggz, z, qn, zs_sum[:N], c16[:N], di8)
  return out.reshape(N)
